# Initial kernel scaffold; baseline (speedup 1.0000x reference)
import jax
import jax.numpy as jnp
from jax.experimental import pallas as pl


def kernel(x, edge_index, batch, W1, b1, W2, b2, W3, b3, Wp1a, bp1a, Wp1b, bp1b, Wp2a, bp2a, Wp2b, bp2b, Wp3a, bp3a, Wp3b, bp3b, p1, p2):
    raise NotImplementedError("write your pallas kernel here")


# speedup vs baseline: 14.8811x; 14.8811x over previous
import functools
import jax
import jax.numpy as jnp
from jax import lax
from jax.experimental import pallas as pl
from jax.experimental.pallas import tpu as pltpu
from jax.experimental.pallas import tpu_sc as plsc

NC, NS, LANES = 2, 16, 16
G = 64
RATIO = 0.5
NEG = -3.0e38


def _mm_body(a_ref, w_ref, b_ref, o_ref, *, act):
    h = jnp.dot(a_ref[...], w_ref[...], preferred_element_type=jnp.float32) + b_ref[...]
    if act:
        h = jnp.maximum(h, 0.0)
    o_ref[...] = h


def _mm_rs_body(a_ref, rs_ref, w_ref, b_ref, o_ref, *, act):
    a = a_ref[...] * rs_ref[...]
    h = jnp.dot(a, w_ref[...], preferred_element_type=jnp.float32) + b_ref[...]
    if act:
        h = jnp.maximum(h, 0.0)
    o_ref[...] = h


def _mm(a, w, b, act=False, rowscale=None, block=1000):
    n, k = a.shape
    m = w.shape[1]
    b2 = b.reshape(1, m)
    if rowscale is None:
        return pl.pallas_call(
            functools.partial(_mm_body, act=act),
            grid=(n // block,),
            in_specs=[pl.BlockSpec((block, k), lambda i: (i, 0)),
                      pl.BlockSpec((k, m), lambda i: (0, 0)),
                      pl.BlockSpec((1, m), lambda i: (0, 0))],
            out_specs=pl.BlockSpec((block, m), lambda i: (i, 0)),
            out_shape=jax.ShapeDtypeStruct((n, m), jnp.float32),
        )(a, w, b2)
    return pl.pallas_call(
        functools.partial(_mm_rs_body, act=act),
        grid=(n // block,),
        in_specs=[pl.BlockSpec((block, k), lambda i: (i, 0)),
                  pl.BlockSpec((block, 1), lambda i: (i, 0)),
                  pl.BlockSpec((k, m), lambda i: (0, 0)),
                  pl.BlockSpec((1, m), lambda i: (0, 0))],
        out_specs=pl.BlockSpec((block, m), lambda i: (i, 0)),
        out_shape=jax.ShapeDtypeStruct((n, m), jnp.float32),
    )(a, rowscale, w, b2)


def _scale_body(hp_ref, d_ref, m_ref, hs_ref, dis_ref):
    d = jnp.sum(d_ref[...], axis=1, keepdims=True)
    m = m_ref[...]
    dis = lax.rsqrt(1.0 + m * d)
    dis_ref[...] = dis
    hs = dis * hp_ref[...]
    h = hs_ref.shape[2]
    hs_ref[0, :, :] = hs[:, :h]
    hs_ref[1, :, :] = hs[:, h:]


def _scale(hp, dparts, m, block=1000):
    n, f = hp.shape
    h = f // 2
    return pl.pallas_call(
        _scale_body,
        grid=(n // block,),
        in_specs=[pl.BlockSpec((block, f), lambda i: (i, 0)),
                  pl.BlockSpec((block, NC * NS), lambda i: (i, 0)),
                  pl.BlockSpec((block, 1), lambda i: (i, 0))],
        out_specs=[pl.BlockSpec((2, block, h), lambda i: (0, i, 0)),
                   pl.BlockSpec((block, 1), lambda i: (i, 0))],
        out_shape=[jax.ShapeDtypeStruct((2, n, h), jnp.float32),
                   jax.ShapeDtypeStruct((n, 1), jnp.float32)],
    )(hp, dparts, m)


def _asm_body(u_ref, hp_ref, dis_ref, m_ref, b_ref, h_ref, x_ref):
    u = jnp.concatenate([u_ref[0], u_ref[1]], axis=1)
    dis = dis_ref[...]
    m = m_ref[...]
    h = (m * dis) * u + (dis * dis) * hp_ref[...] + b_ref[...]
    h_ref[...] = h
    x_ref[...] = jnp.maximum(h, 0.0)


def _assemble(u2, hp, dis, m, b, block=1000):
    n, f = hp.shape
    h = f // 2
    return pl.pallas_call(
        _asm_body,
        grid=(n // block,),
        in_specs=[pl.BlockSpec((2, block, h), lambda i: (0, i, 0)),
                  pl.BlockSpec((block, f), lambda i: (i, 0)),
                  pl.BlockSpec((block, 1), lambda i: (i, 0)),
                  pl.BlockSpec((block, 1), lambda i: (i, 0)),
                  pl.BlockSpec((1, f), lambda i: (0, 0))],
        out_specs=[pl.BlockSpec((block, f), lambda i: (i, 0)),
                   pl.BlockSpec((block, f), lambda i: (i, 0))],
        out_shape=[jax.ShapeDtypeStruct((n, f), jnp.float32),
                   jax.ShapeDtypeStruct((n, f), jnp.float32)],
    )(u2, hp, dis, m, b.reshape(1, f))


def _score_body(x_ref, p_ref, o_ref):
    p = p_ref[...]
    nrm = jnp.sqrt(jnp.sum(p * p))
    s = jnp.sum(x_ref[...] * p, axis=1, keepdims=True) / nrm
    o_ref[...] = jnp.tanh(s)


def _score(x, p, block=1000):
    n, f = x.shape
    return pl.pallas_call(
        _score_body,
        grid=(n // block,),
        in_specs=[pl.BlockSpec((block, f), lambda i: (i, 0)),
                  pl.BlockSpec((1, f), lambda i: (0, 0))],
        out_specs=pl.BlockSpec((block, 1), lambda i: (i, 0)),
        out_shape=jax.ShapeDtypeStruct((n, 1), jnp.float32),
    )(x, p.reshape(1, f))


def _seg_body(x_ref, a_ref, c_ref, bt_ref, incl_ref,
              sx_ref, mx_ref, ctc_ref, ctr_ref, sa_ref, sc_ref):
    i = pl.program_id(0)
    bt = bt_ref[...]
    incl = incl_ref[...]
    gids = lax.broadcasted_iota(jnp.int32, (1, G), 1).astype(jnp.float32)
    oh = jnp.where(bt == gids, incl, 0.0)
    x = x_ref[...]
    dn = (((0,), (0,)), ((), ()))
    sx = lax.dot_general(oh, x, dn, preferred_element_type=jnp.float32)
    sa = lax.dot_general(oh, a_ref[...], dn, preferred_element_type=jnp.float32)
    sc = lax.dot_general(oh, c_ref[...], dn, preferred_element_type=jnp.float32)
    ones = jnp.ones((bt.shape[0], 1), jnp.float32)
    ctc = lax.dot_general(oh, ones, dn, preferred_element_type=jnp.float32)
    ctr = jnp.sum(oh, axis=0, keepdims=True)

    rows = []
    for g in range(G):
        sel = jnp.where((bt == float(g)) & (incl > 0.0), 0.0, NEG)
        rows.append(jnp.max(x + sel, axis=0, keepdims=True))
    mx = jnp.concatenate(rows, axis=0)

    @pl.when(i == 0)
    def _():
        sx_ref[...] = sx
        sa_ref[...] = sa
        sc_ref[...] = sc
        ctc_ref[...] = ctc
        ctr_ref[...] = ctr
        mx_ref[...] = mx

    @pl.when(i > 0)
    def _():
        sx_ref[...] += sx
        sa_ref[...] += sa
        sc_ref[...] += sc
        ctc_ref[...] += ctc
        ctr_ref[...] += ctr
        mx_ref[...] = jnp.maximum(mx_ref[...], mx)


def _segreduce(x, a, c, bt, incl, block=1000):
    n, f = x.shape
    fa = a.shape[1]
    return pl.pallas_call(
        _seg_body,
        grid=(n // block,),
        in_specs=[pl.BlockSpec((block, f), lambda i: (i, 0)),
                  pl.BlockSpec((block, fa), lambda i: (i, 0)),
                  pl.BlockSpec((block, fa), lambda i: (i, 0)),
                  pl.BlockSpec((block, 1), lambda i: (i, 0)),
                  pl.BlockSpec((block, 1), lambda i: (i, 0))],
        out_specs=[pl.BlockSpec((G, f), lambda i: (0, 0)),
                   pl.BlockSpec((G, f), lambda i: (0, 0)),
                   pl.BlockSpec((G, 1), lambda i: (0, 0)),
                   pl.BlockSpec((1, G), lambda i: (0, 0)),
                   pl.BlockSpec((G, fa), lambda i: (0, 0)),
                   pl.BlockSpec((G, fa), lambda i: (0, 0))],
        out_shape=[jax.ShapeDtypeStruct((G, f), jnp.float32),
                   jax.ShapeDtypeStruct((G, f), jnp.float32),
                   jax.ShapeDtypeStruct((G, 1), jnp.float32),
                   jax.ShapeDtypeStruct((1, G), jnp.float32),
                   jax.ShapeDtypeStruct((G, fa), jnp.float32),
                   jax.ShapeDtypeStruct((G, fa), jnp.float32)],
    )(x, a, c, bt, incl)


def _topk_body(s_ref, bt_ref, act_ref, sT_ref, btT_ref, actT_ref, cnt_ref,
               mask_ref, ms_ref, *, block, cblock, npad):
    i = pl.program_id(0)
    s = s_ref[...]
    bt = bt_ref[...]
    act = act_ref[...]
    ridx = (i * block).astype(jnp.float32) + lax.broadcasted_iota(jnp.int32, (block, 1), 0).astype(jnp.float32)

    rank = jnp.zeros((block, 1), jnp.float32)
    for j in range(npad // cblock):
        cs = sT_ref[0:1, j * cblock:(j + 1) * cblock]
        cb = btT_ref[0:1, j * cblock:(j + 1) * cblock]
        ca = actT_ref[0:1, j * cblock:(j + 1) * cblock]
        cidx = float(j * cblock) + lax.broadcasted_iota(jnp.int32, (1, cblock), 1).astype(jnp.float32)
        beats = (cs > s) | ((cs == s) & (cidx < ridx))
        cmp = jnp.where((cb == bt) & (ca > 0.0) & beats, 1.0, 0.0)
        rank = rank + jnp.sum(cmp, axis=1, keepdims=True)

    gids = lax.broadcasted_iota(jnp.int32, (1, G), 1).astype(jnp.float32)
    k = jnp.ceil(RATIO * cnt_ref[...])
    krow = jnp.sum(jnp.where(bt == gids, k, 0.0), axis=1, keepdims=True)
    mask = jnp.where((act > 0.0) & (rank < krow), 1.0, 0.0)
    mask_ref[...] = mask
    ms_ref[...] = mask * s


def _topk(s, bt, act, sT, btT, actT, cnt, block=1000, cblock=2048):
    n = s.shape[0]
    npad = sT.shape[1]
    return pl.pallas_call(
        functools.partial(_topk_body, block=block, cblock=cblock, npad=npad),
        grid=(n // block,),
        in_specs=[pl.BlockSpec((block, 1), lambda i: (i, 0)),
                  pl.BlockSpec((block, 1), lambda i: (i, 0)),
                  pl.BlockSpec((block, 1), lambda i: (i, 0)),
                  pl.BlockSpec((1, npad), lambda i: (0, 0)),
                  pl.BlockSpec((1, npad), lambda i: (0, 0)),
                  pl.BlockSpec((1, npad), lambda i: (0, 0)),
                  pl.BlockSpec((1, G), lambda i: (0, 0))],
        out_specs=[pl.BlockSpec((block, 1), lambda i: (i, 0)),
                   pl.BlockSpec((block, 1), lambda i: (i, 0))],
        out_shape=[jax.ShapeDtypeStruct((n, 1), jnp.float32),
                   jax.ShapeDtypeStruct((n, 1), jnp.float32)],
    )(s, bt, act, sT, btT, actT, cnt)


def _fin_body(sx1, mx1, ct1, sx2, mx2, ct2, sx3, mx3, ct3, o_ref):
    f = sx1.shape[1]

    def gpool(sx_ref, mx_ref, ct_ref):
        cnt = ct_ref[...]
        mean = sx_ref[...] / jnp.maximum(cnt, 1.0)
        mx = jnp.where(cnt > 0.0, mx_ref[...], 0.0)
        return mx, mean
    a1, b1 = gpool(sx1, mx1, ct1)
    a2, b2 = gpool(sx2, mx2, ct2)
    a3, b3 = gpool(sx3, mx3, ct3)
    o_ref[:, :f] = a1 + a2 + a3
    o_ref[:, f:] = b1 + b2 + b3


def _finalize(parts):
    f = parts[0].shape[1]
    return pl.pallas_call(
        _fin_body,
        out_shape=jax.ShapeDtypeStruct((G, 2 * f), jnp.float32),
    )(*parts)


def _sc_mesh():
    return plsc.VectorSubcoreMesh(core_axis_name="c", subcore_axis_name="s",
                                  num_cores=NC, num_subcores=NS)


@functools.lru_cache(maxsize=None)
def _make_deg(n, e):
    ew = e // (NC * NS)

    def body(src_hbm, dst_hbm, m_hbm, d_out, m_v, d_v, s_v, t_v):
        wid = lax.axis_index("s") * NC + lax.axis_index("c")
        pltpu.sync_copy(m_hbm, m_v)
        pltpu.sync_copy(src_hbm.at[pl.ds(wid * ew, ew)], s_v)
        pltpu.sync_copy(dst_hbm.at[pl.ds(wid * ew, ew)], t_v)

        def zero(i, carry):
            d_v[pl.ds(i * LANES, LANES)] = jnp.zeros((LANES,), jnp.float32)
            return carry
        lax.fori_loop(0, n // LANES, zero, 0)

        def step(i, carry):
            sidx = s_v[pl.ds(i * LANES, LANES)]
            didx = t_v[pl.ds(i * LANES, LANES)]
            vals = plsc.load_gather(m_v, [sidx])
            plsc.addupdate_scatter(d_v, [didx], vals)
            return carry
        lax.fori_loop(0, ew // LANES, step, 0)
        pltpu.sync_copy(d_v, d_out.at[wid])

    return pl.kernel(
        body,
        out_type=jax.ShapeDtypeStruct((NC * NS, n), jnp.float32),
        mesh=_sc_mesh(),
        compiler_params=pltpu.CompilerParams(needs_layout_passes=False),
        scratch_types=[pltpu.VMEM((n,), jnp.float32),
                       pltpu.VMEM((n,), jnp.float32),
                       pltpu.VMEM((ew,), jnp.int32),
                       pltpu.VMEM((ew,), jnp.int32)],
    )


CHUNK = 125


@functools.lru_cache(maxsize=None)
def _make_spmm(n, e, h):
    chw = e // (NS * CHUNK)
    npad = _npad(n)
    stripe = npad // NS

    grp = 8

    def body(hs_hbm, src_hbm, dst_hbm, z_hbm, u_out, s_v, t_v, rows_v, u_sh, sem):
        cid = lax.axis_index("c")
        sid = lax.axis_index("s")
        pltpu.sync_copy(z_hbm, u_sh.at[pl.ds(sid * stripe, stripe)])
        plsc.subcore_barrier()

        def group(j, carry):
            base = sid * chw + j * grp
            pltpu.sync_copy(src_hbm.at[cid].at[pl.ds(base, grp)], s_v)
            pltpu.sync_copy(dst_hbm.at[pl.ds(base, grp)], t_v)
            for r in range(grp):
                pltpu.async_copy(hs_hbm.at[s_v.at[r]], rows_v, sem).wait()
                pltpu.sync_copy(rows_v, u_sh.at[t_v.at[r]], add=True)
            return carry
        lax.fori_loop(0, chw // grp, group, 0)
        plsc.subcore_barrier()
        pltpu.sync_copy(u_sh.at[pl.ds(sid * stripe, stripe)],
                        u_out.at[cid].at[pl.ds(sid * stripe, stripe)])

    return pl.kernel(
        body,
        out_type=jax.ShapeDtypeStruct((NC, npad, h), jnp.float32),
        mesh=_sc_mesh(),
        compiler_params=pltpu.CompilerParams(needs_layout_passes=False),
        scratch_types=[pltpu.VMEM((grp, CHUNK), jnp.int32),
                       pltpu.VMEM((grp, CHUNK), jnp.int32),
                       pltpu.VMEM((CHUNK, h), jnp.float32),
                       pltpu.VMEM_SHARED((npad, h), jnp.float32),
                       pltpu.SemaphoreType.DMA],
    )


def _npad(n):
    q = NS * 8
    return ((n + q - 1) // q) * q


def kernel(x, edge_index, batch, W1, b1, W2, b2, W3, b3, Wp1a, bp1a, Wp1b, bp1b,
           Wp2a, bp2a, Wp2b, bp2b, Wp3a, bp3a, Wp3b, bp3b, p1, p2):
    n = x.shape[0]
    e = edge_index.shape[1]
    hh = W1.shape[1]
    half = hh // 2

    src = edge_index[0]
    dst = edge_index[1]
    src2 = jnp.stack([src, src + n]).reshape(2, e // CHUNK, CHUNK)
    dst2 = dst.reshape(e // CHUNK, CHUNK)
    zrows = jnp.zeros((_npad(n) // NS, half), jnp.float32)
    ones_col = jnp.ones((n, 1), jnp.float32)
    btf = batch.astype(jnp.float32).reshape(n, 1)
    npad = ((n + 2047) // 2048) * 2048
    pad = npad - n

    def rowT(v, fill):
        return jnp.pad(v.reshape(1, n), ((0, 0), (0, pad)), constant_values=fill)
    btfT = rowT(btf, -1.0)
    onesT = rowT(ones_col, 0.0)

    deg_k = _make_deg(n, e)
    spmm_k = _make_spmm(n, e, half)

    def gcn_layer(feat, rowscale, m_col, W, b):
        m1d = m_col.reshape(n)
        dparts = deg_k(src, dst, m1d)
        hp = _mm(feat, W, jnp.zeros_like(b), rowscale=rowscale)
        hs2, dis = _scale(hp, dparts.T, m_col)
        u2 = spmm_k(hs2.reshape(2 * n, half), src2, dst2, zrows)
        return _assemble(u2, hp, dis, m_col, b)

    def mlp(v, wa, ba, wb, bb):
        return _mm(_mm(v, wa, ba, act=True), wb, bb)

    h1, x1 = gcn_layer(x, None, ones_col, W1, b1)
    seg1 = _segreduce(x1, mlp(h1, Wp1a, bp1a, Wp1b, bp1b),
                      mlp(x1, Wp1a, bp1a, Wp1b, bp1b), btf, ones_col)
    sx1, mxx1, ctc1, ctr1, g0_1, proj_1 = seg1
    s1 = _score(x1, p1)
    mask1, ms1 = _topk(s1, btf, ones_col, rowT(s1, 0.0), btfT, onesT, ctr1)

    h2, x2 = gcn_layer(x1, ms1, mask1, W2, b2)
    seg2 = _segreduce(x2, mlp(h2, Wp2a, bp2a, Wp2b, bp2b),
                      mlp(x2, Wp2a, bp2a, Wp2b, bp2b), btf, mask1)
    sx2, mxx2, ctc2, ctr2, g1_1, proj_2 = seg2
    s2 = _score(x2, p2)
    mask2, ms2 = _topk(s2, btf, mask1, rowT(s2, 0.0), btfT,
                       rowT(mask1, 0.0), ctr2)

    h3, x3 = gcn_layer(x2, ms2, mask2, W3, b3)
    seg3 = _segreduce(x3, mlp(h3, Wp3a, bp3a, Wp3b, bp3b),
                      mlp(x3, Wp3a, bp3a, Wp3b, bp3b), btf, mask2)
    sx3, mxx3, ctc3, ctr3, g2_1, proj_3 = seg3

    out = _finalize((sx1, mxx1, ctc1, sx2, mxx2, ctc2, sx3, mxx3, ctc3))
    return (out, proj_1, proj_2, proj_3, g0_1, g0_1, g1_1, g1_1, g2_1, g2_1)


# speedup vs baseline: 14.8869x; 1.0004x over previous
import functools
import jax
import jax.numpy as jnp
from jax import lax
from jax.experimental import pallas as pl
from jax.experimental.pallas import tpu as pltpu
from jax.experimental.pallas import tpu_sc as plsc

NC, NS, LANES = 2, 16, 16
G = 64
RATIO = 0.5
NEG = -3.0e38


def _bdot(a, w):
    return jnp.dot(a.astype(jnp.bfloat16), w.astype(jnp.bfloat16),
                   preferred_element_type=jnp.float32)


def _mm_body(a_ref, w_ref, b_ref, o_ref, *, act):
    h = _bdot(a_ref[...], w_ref[...]) + b_ref[...]
    if act:
        h = jnp.maximum(h, 0.0)
    o_ref[...] = h


def _mm_rs_body(a_ref, rs_ref, w_ref, b_ref, o_ref, *, act):
    a = a_ref[...] * rs_ref[...]
    h = _bdot(a, w_ref[...]) + b_ref[...]
    if act:
        h = jnp.maximum(h, 0.0)
    o_ref[...] = h


def _mm(a, w, b, act=False, rowscale=None, block=1000):
    n, k = a.shape
    m = w.shape[1]
    b2 = b.reshape(1, m)
    if rowscale is None:
        return pl.pallas_call(
            functools.partial(_mm_body, act=act),
            grid=(n // block,),
            in_specs=[pl.BlockSpec((block, k), lambda i: (i, 0)),
                      pl.BlockSpec((k, m), lambda i: (0, 0)),
                      pl.BlockSpec((1, m), lambda i: (0, 0))],
            out_specs=pl.BlockSpec((block, m), lambda i: (i, 0)),
            out_shape=jax.ShapeDtypeStruct((n, m), jnp.float32),
        )(a, w, b2)
    return pl.pallas_call(
        functools.partial(_mm_rs_body, act=act),
        grid=(n // block,),
        in_specs=[pl.BlockSpec((block, k), lambda i: (i, 0)),
                  pl.BlockSpec((block, 1), lambda i: (i, 0)),
                  pl.BlockSpec((k, m), lambda i: (0, 0)),
                  pl.BlockSpec((1, m), lambda i: (0, 0))],
        out_specs=pl.BlockSpec((block, m), lambda i: (i, 0)),
        out_shape=jax.ShapeDtypeStruct((n, m), jnp.float32),
    )(a, rowscale, w, b2)


def _scale_body(hp_ref, d_ref, m_ref, hs_ref, dis_ref):
    d = jnp.sum(d_ref[...], axis=1, keepdims=True)
    m = m_ref[...]
    dis = lax.rsqrt(1.0 + m * d)
    dis_ref[...] = dis
    hs = dis * hp_ref[...]
    h = hs_ref.shape[2]
    hs_ref[0, :, :] = hs[:, :h]
    hs_ref[1, :, :] = hs[:, h:]


def _scale(hp, dparts, m, block=1000):
    n, f = hp.shape
    h = f // 2
    return pl.pallas_call(
        _scale_body,
        grid=(n // block,),
        in_specs=[pl.BlockSpec((block, f), lambda i: (i, 0)),
                  pl.BlockSpec((block, NC * NS), lambda i: (i, 0)),
                  pl.BlockSpec((block, 1), lambda i: (i, 0))],
        out_specs=[pl.BlockSpec((2, block, h), lambda i: (0, i, 0)),
                   pl.BlockSpec((block, 1), lambda i: (i, 0))],
        out_shape=[jax.ShapeDtypeStruct((2, n, h), jnp.float32),
                   jax.ShapeDtypeStruct((n, 1), jnp.float32)],
    )(hp, dparts, m)


def _asm_body(u_ref, hp_ref, dis_ref, m_ref, b_ref, h_ref, x_ref):
    u = jnp.concatenate([u_ref[0], u_ref[1]], axis=1)
    dis = dis_ref[...]
    m = m_ref[...]
    h = (m * dis) * u + (dis * dis) * hp_ref[...] + b_ref[...]
    h_ref[...] = h
    x_ref[...] = jnp.maximum(h, 0.0)


def _assemble(u2, hp, dis, m, b, block=1000):
    n, f = hp.shape
    h = f // 2
    return pl.pallas_call(
        _asm_body,
        grid=(n // block,),
        in_specs=[pl.BlockSpec((2, block, h), lambda i: (0, i, 0)),
                  pl.BlockSpec((block, f), lambda i: (i, 0)),
                  pl.BlockSpec((block, 1), lambda i: (i, 0)),
                  pl.BlockSpec((block, 1), lambda i: (i, 0)),
                  pl.BlockSpec((1, f), lambda i: (0, 0))],
        out_specs=[pl.BlockSpec((block, f), lambda i: (i, 0)),
                   pl.BlockSpec((block, f), lambda i: (i, 0))],
        out_shape=[jax.ShapeDtypeStruct((n, f), jnp.float32),
                   jax.ShapeDtypeStruct((n, f), jnp.float32)],
    )(u2, hp, dis, m, b.reshape(1, f))


def _score_body(x_ref, p_ref, o_ref):
    p = p_ref[...]
    nrm = jnp.sqrt(jnp.sum(p * p))
    xb = x_ref[...].astype(jnp.bfloat16).astype(jnp.float32)
    pb = p.astype(jnp.bfloat16).astype(jnp.float32)
    s = jnp.sum(xb * pb, axis=1, keepdims=True) / nrm
    o_ref[...] = jnp.tanh(s)


def _score(x, p, block=1000):
    n, f = x.shape
    return pl.pallas_call(
        _score_body,
        grid=(n // block,),
        in_specs=[pl.BlockSpec((block, f), lambda i: (i, 0)),
                  pl.BlockSpec((1, f), lambda i: (0, 0))],
        out_specs=pl.BlockSpec((block, 1), lambda i: (i, 0)),
        out_shape=jax.ShapeDtypeStruct((n, 1), jnp.float32),
    )(x, p.reshape(1, f))


def _seg_body(x_ref, a_ref, c_ref, bt_ref, incl_ref,
              sx_ref, mx_ref, ctc_ref, ctr_ref, sa_ref, sc_ref):
    i = pl.program_id(0)
    bt = bt_ref[...]
    incl = incl_ref[...]
    gids = lax.broadcasted_iota(jnp.int32, (1, G), 1).astype(jnp.float32)
    oh = jnp.where(bt == gids, incl, 0.0)
    x = x_ref[...]
    dn = (((0,), (0,)), ((), ()))
    sx = lax.dot_general(oh, x, dn, preferred_element_type=jnp.float32)
    sa = lax.dot_general(oh, a_ref[...], dn, preferred_element_type=jnp.float32)
    sc = lax.dot_general(oh, c_ref[...], dn, preferred_element_type=jnp.float32)
    ones = jnp.ones((bt.shape[0], 1), jnp.float32)
    ctc = lax.dot_general(oh, ones, dn, preferred_element_type=jnp.float32)
    ctr = jnp.sum(oh, axis=0, keepdims=True)

    rows = []
    for g in range(G):
        sel = jnp.where((bt == float(g)) & (incl > 0.0), 0.0, NEG)
        rows.append(jnp.max(x + sel, axis=0, keepdims=True))
    mx = jnp.concatenate(rows, axis=0)

    @pl.when(i == 0)
    def _():
        sx_ref[...] = sx
        sa_ref[...] = sa
        sc_ref[...] = sc
        ctc_ref[...] = ctc
        ctr_ref[...] = ctr
        mx_ref[...] = mx

    @pl.when(i > 0)
    def _():
        sx_ref[...] += sx
        sa_ref[...] += sa
        sc_ref[...] += sc
        ctc_ref[...] += ctc
        ctr_ref[...] += ctr
        mx_ref[...] = jnp.maximum(mx_ref[...], mx)


def _segreduce(x, a, c, bt, incl, block=1000):
    n, f = x.shape
    fa = a.shape[1]
    return pl.pallas_call(
        _seg_body,
        grid=(n // block,),
        in_specs=[pl.BlockSpec((block, f), lambda i: (i, 0)),
                  pl.BlockSpec((block, fa), lambda i: (i, 0)),
                  pl.BlockSpec((block, fa), lambda i: (i, 0)),
                  pl.BlockSpec((block, 1), lambda i: (i, 0)),
                  pl.BlockSpec((block, 1), lambda i: (i, 0))],
        out_specs=[pl.BlockSpec((G, f), lambda i: (0, 0)),
                   pl.BlockSpec((G, f), lambda i: (0, 0)),
                   pl.BlockSpec((G, 1), lambda i: (0, 0)),
                   pl.BlockSpec((1, G), lambda i: (0, 0)),
                   pl.BlockSpec((G, fa), lambda i: (0, 0)),
                   pl.BlockSpec((G, fa), lambda i: (0, 0))],
        out_shape=[jax.ShapeDtypeStruct((G, f), jnp.float32),
                   jax.ShapeDtypeStruct((G, f), jnp.float32),
                   jax.ShapeDtypeStruct((G, 1), jnp.float32),
                   jax.ShapeDtypeStruct((1, G), jnp.float32),
                   jax.ShapeDtypeStruct((G, fa), jnp.float32),
                   jax.ShapeDtypeStruct((G, fa), jnp.float32)],
    )(x, a, c, bt, incl)


def _topk_body(s_ref, bt_ref, act_ref, sT_ref, btT_ref, actT_ref, cnt_ref,
               mask_ref, ms_ref, *, block, cblock, npad):
    i = pl.program_id(0)
    s = s_ref[...]
    bt = bt_ref[...]
    act = act_ref[...]
    ridx = (i * block).astype(jnp.float32) + lax.broadcasted_iota(jnp.int32, (block, 1), 0).astype(jnp.float32)

    rank = jnp.zeros((block, 1), jnp.float32)
    for j in range(npad // cblock):
        cs = sT_ref[0:1, j * cblock:(j + 1) * cblock]
        cb = btT_ref[0:1, j * cblock:(j + 1) * cblock]
        ca = actT_ref[0:1, j * cblock:(j + 1) * cblock]
        cidx = float(j * cblock) + lax.broadcasted_iota(jnp.int32, (1, cblock), 1).astype(jnp.float32)
        beats = (cs > s) | ((cs == s) & (cidx < ridx))
        cmp = jnp.where((cb == bt) & (ca > 0.0) & beats, 1.0, 0.0)
        rank = rank + jnp.sum(cmp, axis=1, keepdims=True)

    gids = lax.broadcasted_iota(jnp.int32, (1, G), 1).astype(jnp.float32)
    k = jnp.ceil(RATIO * cnt_ref[...])
    krow = jnp.sum(jnp.where(bt == gids, k, 0.0), axis=1, keepdims=True)
    mask = jnp.where((act > 0.0) & (rank < krow), 1.0, 0.0)
    mask_ref[...] = mask
    ms_ref[...] = mask * s


def _topk(s, bt, act, sT, btT, actT, cnt, block=1000, cblock=2048):
    n = s.shape[0]
    npad = sT.shape[1]
    return pl.pallas_call(
        functools.partial(_topk_body, block=block, cblock=cblock, npad=npad),
        grid=(n // block,),
        in_specs=[pl.BlockSpec((block, 1), lambda i: (i, 0)),
                  pl.BlockSpec((block, 1), lambda i: (i, 0)),
                  pl.BlockSpec((block, 1), lambda i: (i, 0)),
                  pl.BlockSpec((1, npad), lambda i: (0, 0)),
                  pl.BlockSpec((1, npad), lambda i: (0, 0)),
                  pl.BlockSpec((1, npad), lambda i: (0, 0)),
                  pl.BlockSpec((1, G), lambda i: (0, 0))],
        out_specs=[pl.BlockSpec((block, 1), lambda i: (i, 0)),
                   pl.BlockSpec((block, 1), lambda i: (i, 0))],
        out_shape=[jax.ShapeDtypeStruct((n, 1), jnp.float32),
                   jax.ShapeDtypeStruct((n, 1), jnp.float32)],
    )(s, bt, act, sT, btT, actT, cnt)


def _fin_body(sx1, mx1, ct1, sx2, mx2, ct2, sx3, mx3, ct3, o_ref):
    f = sx1.shape[1]

    def gpool(sx_ref, mx_ref, ct_ref):
        cnt = ct_ref[...]
        mean = sx_ref[...] / jnp.maximum(cnt, 1.0)
        mx = jnp.where(cnt > 0.0, mx_ref[...], 0.0)
        return mx, mean
    a1, b1 = gpool(sx1, mx1, ct1)
    a2, b2 = gpool(sx2, mx2, ct2)
    a3, b3 = gpool(sx3, mx3, ct3)
    o_ref[:, :f] = a1 + a2 + a3
    o_ref[:, f:] = b1 + b2 + b3


def _finalize(parts):
    f = parts[0].shape[1]
    return pl.pallas_call(
        _fin_body,
        out_shape=jax.ShapeDtypeStruct((G, 2 * f), jnp.float32),
    )(*parts)


def _sc_mesh():
    return plsc.VectorSubcoreMesh(core_axis_name="c", subcore_axis_name="s",
                                  num_cores=NC, num_subcores=NS)


@functools.lru_cache(maxsize=None)
def _make_deg(n, e):
    ew = e // (NC * NS)

    def body(src_hbm, dst_hbm, m_hbm, d_out, m_v, d_v, s_v, t_v):
        wid = lax.axis_index("s") * NC + lax.axis_index("c")
        pltpu.sync_copy(m_hbm, m_v)
        pltpu.sync_copy(src_hbm.at[pl.ds(wid * ew, ew)], s_v)
        pltpu.sync_copy(dst_hbm.at[pl.ds(wid * ew, ew)], t_v)

        def zero(i, carry):
            d_v[pl.ds(i * LANES, LANES)] = jnp.zeros((LANES,), jnp.float32)
            return carry
        lax.fori_loop(0, n // LANES, zero, 0)

        def step(i, carry):
            sidx = s_v[pl.ds(i * LANES, LANES)]
            didx = t_v[pl.ds(i * LANES, LANES)]
            vals = plsc.load_gather(m_v, [sidx])
            plsc.addupdate_scatter(d_v, [didx], vals)
            return carry
        lax.fori_loop(0, ew // LANES, step, 0)
        pltpu.sync_copy(d_v, d_out.at[wid])

    return pl.kernel(
        body,
        out_type=jax.ShapeDtypeStruct((NC * NS, n), jnp.float32),
        mesh=_sc_mesh(),
        compiler_params=pltpu.CompilerParams(needs_layout_passes=False),
        scratch_types=[pltpu.VMEM((n,), jnp.float32),
                       pltpu.VMEM((n,), jnp.float32),
                       pltpu.VMEM((ew,), jnp.int32),
                       pltpu.VMEM((ew,), jnp.int32)],
    )


CHUNK = 125


@functools.lru_cache(maxsize=None)
def _make_spmm(n, e, h):
    chw = e // (NS * CHUNK)
    npad = _npad(n)
    stripe = npad // NS

    grp = 8

    def body(hs_hbm, src_hbm, dst_hbm, z_hbm, u_out, s_v, t_v, rows_v, u_sh, sem):
        cid = lax.axis_index("c")
        sid = lax.axis_index("s")
        pltpu.sync_copy(z_hbm, u_sh.at[pl.ds(sid * stripe, stripe)])
        plsc.subcore_barrier()

        def group(j, carry):
            base = sid * chw + j * grp
            pltpu.sync_copy(src_hbm.at[cid].at[pl.ds(base, grp)], s_v)
            pltpu.sync_copy(dst_hbm.at[pl.ds(base, grp)], t_v)
            for r in range(grp):
                pltpu.async_copy(hs_hbm.at[s_v.at[r]], rows_v, sem).wait()
                pltpu.sync_copy(rows_v, u_sh.at[t_v.at[r]], add=True)
            return carry
        lax.fori_loop(0, chw // grp, group, 0)
        plsc.subcore_barrier()
        pltpu.sync_copy(u_sh.at[pl.ds(sid * stripe, stripe)],
                        u_out.at[cid].at[pl.ds(sid * stripe, stripe)])

    return pl.kernel(
        body,
        out_type=jax.ShapeDtypeStruct((NC, npad, h), jnp.float32),
        mesh=_sc_mesh(),
        compiler_params=pltpu.CompilerParams(needs_layout_passes=False),
        scratch_types=[pltpu.VMEM((grp, CHUNK), jnp.int32),
                       pltpu.VMEM((grp, CHUNK), jnp.int32),
                       pltpu.VMEM((CHUNK, h), jnp.float32),
                       pltpu.VMEM_SHARED((npad, h), jnp.float32),
                       pltpu.SemaphoreType.DMA],
    )


def _npad(n):
    q = NS * 8
    return ((n + q - 1) // q) * q


def kernel(x, edge_index, batch, W1, b1, W2, b2, W3, b3, Wp1a, bp1a, Wp1b, bp1b,
           Wp2a, bp2a, Wp2b, bp2b, Wp3a, bp3a, Wp3b, bp3b, p1, p2):
    n = x.shape[0]
    e = edge_index.shape[1]
    hh = W1.shape[1]
    half = hh // 2

    src = edge_index[0]
    dst = edge_index[1]
    src2 = jnp.stack([src, src + n]).reshape(2, e // CHUNK, CHUNK)
    dst2 = dst.reshape(e // CHUNK, CHUNK)
    zrows = jnp.zeros((_npad(n) // NS, half), jnp.float32)
    ones_col = jnp.ones((n, 1), jnp.float32)
    btf = batch.astype(jnp.float32).reshape(n, 1)
    npad = ((n + 2047) // 2048) * 2048
    pad = npad - n

    def rowT(v, fill):
        return jnp.pad(v.reshape(1, n), ((0, 0), (0, pad)), constant_values=fill)
    btfT = rowT(btf, -1.0)
    onesT = rowT(ones_col, 0.0)

    deg_k = _make_deg(n, e)
    spmm_k = _make_spmm(n, e, half)

    def gcn_layer(feat, rowscale, m_col, W, b):
        m1d = m_col.reshape(n)
        dparts = deg_k(src, dst, m1d)
        hp = _mm(feat, W, jnp.zeros_like(b), rowscale=rowscale)
        hs2, dis = _scale(hp, dparts.T, m_col)
        u2 = spmm_k(hs2.reshape(2 * n, half), src2, dst2, zrows)
        return _assemble(u2, hp, dis, m_col, b)

    def mlp(v, wa, ba, wb, bb):
        return _mm(_mm(v, wa, ba, act=True), wb, bb)

    h1, x1 = gcn_layer(x, None, ones_col, W1, b1)
    seg1 = _segreduce(x1, mlp(h1, Wp1a, bp1a, Wp1b, bp1b),
                      mlp(x1, Wp1a, bp1a, Wp1b, bp1b), btf, ones_col)
    sx1, mxx1, ctc1, ctr1, g0_1, proj_1 = seg1
    s1 = _score(x1, p1)
    mask1, ms1 = _topk(s1, btf, ones_col, rowT(s1, 0.0), btfT, onesT, ctr1)

    h2, x2 = gcn_layer(x1, ms1, mask1, W2, b2)
    seg2 = _segreduce(x2, mlp(h2, Wp2a, bp2a, Wp2b, bp2b),
                      mlp(x2, Wp2a, bp2a, Wp2b, bp2b), btf, mask1)
    sx2, mxx2, ctc2, ctr2, g1_1, proj_2 = seg2
    s2 = _score(x2, p2)
    mask2, ms2 = _topk(s2, btf, mask1, rowT(s2, 0.0), btfT,
                       rowT(mask1, 0.0), ctr2)

    h3, x3 = gcn_layer(x2, ms2, mask2, W3, b3)
    seg3 = _segreduce(x3, mlp(h3, Wp3a, bp3a, Wp3b, bp3b),
                      mlp(x3, Wp3a, bp3a, Wp3b, bp3b), btf, mask2)
    sx3, mxx3, ctc3, ctr3, g2_1, proj_3 = seg3

    out = _finalize((sx1, mxx1, ctc1, sx2, mxx2, ctc2, sx3, mxx3, ctc3))
    return (out, proj_1, proj_2, proj_3, g0_1, g0_1, g1_1, g1_1, g2_1, g2_1)


# speedup vs baseline: 16.7210x; 1.1232x over previous
import functools
import jax
import jax.numpy as jnp
from jax import lax
from jax.experimental import pallas as pl
from jax.experimental.pallas import tpu as pltpu
from jax.experimental.pallas import tpu_sc as plsc

NC, NS, LANES = 2, 16, 16
G = 64
RATIO = 0.5
NEG = -3.0e38


def _bdot(a, w):
    return jnp.dot(a.astype(jnp.bfloat16), w.astype(jnp.bfloat16),
                   preferred_element_type=jnp.float32)


def _mm_body(a_ref, w_ref, b_ref, o_ref, *, act):
    h = _bdot(a_ref[...], w_ref[...]) + b_ref[...]
    if act:
        h = jnp.maximum(h, 0.0)
    o_ref[...] = h


def _mm_rs_body(a_ref, rs_ref, w_ref, b_ref, o_ref, *, act):
    a = a_ref[...] * rs_ref[...]
    h = _bdot(a, w_ref[...]) + b_ref[...]
    if act:
        h = jnp.maximum(h, 0.0)
    o_ref[...] = h


def _mm(a, w, b, act=False, rowscale=None, block=1000):
    n, k = a.shape
    m = w.shape[1]
    b2 = b.reshape(1, m)
    if rowscale is None:
        return pl.pallas_call(
            functools.partial(_mm_body, act=act),
            grid=(n // block,),
            in_specs=[pl.BlockSpec((block, k), lambda i: (i, 0)),
                      pl.BlockSpec((k, m), lambda i: (0, 0)),
                      pl.BlockSpec((1, m), lambda i: (0, 0))],
            out_specs=pl.BlockSpec((block, m), lambda i: (i, 0)),
            out_shape=jax.ShapeDtypeStruct((n, m), jnp.float32),
        )(a, w, b2)
    return pl.pallas_call(
        functools.partial(_mm_rs_body, act=act),
        grid=(n // block,),
        in_specs=[pl.BlockSpec((block, k), lambda i: (i, 0)),
                  pl.BlockSpec((block, 1), lambda i: (i, 0)),
                  pl.BlockSpec((k, m), lambda i: (0, 0)),
                  pl.BlockSpec((1, m), lambda i: (0, 0))],
        out_specs=pl.BlockSpec((block, m), lambda i: (i, 0)),
        out_shape=jax.ShapeDtypeStruct((n, m), jnp.float32),
    )(a, rowscale, w, b2)


def _scale_body(hp_ref, d_ref, m_ref, hs_ref, dis_ref):
    d = jnp.sum(d_ref[...], axis=1, keepdims=True)
    m = m_ref[...]
    dis = lax.rsqrt(1.0 + m * d)
    dis_ref[...] = dis
    hs = dis * hp_ref[...]
    h = hs_ref.shape[2]
    hs_ref[0, :, :] = hs[:, :h]
    hs_ref[1, :, :] = hs[:, h:]


def _scale(hp, dparts, m, block=1000):
    n, f = hp.shape
    h = f // 2
    return pl.pallas_call(
        _scale_body,
        grid=(n // block,),
        in_specs=[pl.BlockSpec((block, f), lambda i: (i, 0)),
                  pl.BlockSpec((block, NC * NS), lambda i: (i, 0)),
                  pl.BlockSpec((block, 1), lambda i: (i, 0))],
        out_specs=[pl.BlockSpec((2, block, h), lambda i: (0, i, 0)),
                   pl.BlockSpec((block, 1), lambda i: (i, 0))],
        out_shape=[jax.ShapeDtypeStruct((2, n, h), jnp.float32),
                   jax.ShapeDtypeStruct((n, 1), jnp.float32)],
    )(hp, dparts, m)


def _asm_body(u_ref, hp_ref, dis_ref, m_ref, b_ref, h_ref, x_ref):
    u = jnp.concatenate([u_ref[0], u_ref[1]], axis=1)
    dis = dis_ref[...]
    m = m_ref[...]
    h = (m * dis) * u + (dis * dis) * hp_ref[...] + b_ref[...]
    h_ref[...] = h
    x_ref[...] = jnp.maximum(h, 0.0)


def _assemble(u2, hp, dis, m, b, block=1000):
    n, f = hp.shape
    h = f // 2
    return pl.pallas_call(
        _asm_body,
        grid=(n // block,),
        in_specs=[pl.BlockSpec((2, block, h), lambda i: (0, i, 0)),
                  pl.BlockSpec((block, f), lambda i: (i, 0)),
                  pl.BlockSpec((block, 1), lambda i: (i, 0)),
                  pl.BlockSpec((block, 1), lambda i: (i, 0)),
                  pl.BlockSpec((1, f), lambda i: (0, 0))],
        out_specs=[pl.BlockSpec((block, f), lambda i: (i, 0)),
                   pl.BlockSpec((block, f), lambda i: (i, 0))],
        out_shape=[jax.ShapeDtypeStruct((n, f), jnp.float32),
                   jax.ShapeDtypeStruct((n, f), jnp.float32)],
    )(u2, hp, dis, m, b.reshape(1, f))


def _score_body(x_ref, p_ref, o_ref):
    p = p_ref[...]
    nrm = jnp.sqrt(jnp.sum(p * p))
    xb = x_ref[...].astype(jnp.bfloat16).astype(jnp.float32)
    pb = p.astype(jnp.bfloat16).astype(jnp.float32)
    s = jnp.sum(xb * pb, axis=1, keepdims=True) / nrm
    o_ref[...] = jnp.tanh(s)


def _score(x, p, block=1000):
    n, f = x.shape
    return pl.pallas_call(
        _score_body,
        grid=(n // block,),
        in_specs=[pl.BlockSpec((block, f), lambda i: (i, 0)),
                  pl.BlockSpec((1, f), lambda i: (0, 0))],
        out_specs=pl.BlockSpec((block, 1), lambda i: (i, 0)),
        out_shape=jax.ShapeDtypeStruct((n, 1), jnp.float32),
    )(x, p.reshape(1, f))


def _seg_body(x_ref, a_ref, c_ref, bt_ref, incl_ref,
              sx_ref, mx_ref, ctc_ref, ctr_ref, sa_ref, sc_ref):
    i = pl.program_id(0)
    bt = bt_ref[...]
    incl = incl_ref[...]
    gids = lax.broadcasted_iota(jnp.int32, (1, G), 1).astype(jnp.float32)
    oh = jnp.where(bt == gids, incl, 0.0)
    x = x_ref[...]
    dn = (((0,), (0,)), ((), ()))
    sx = lax.dot_general(oh, x, dn, preferred_element_type=jnp.float32)
    sa = lax.dot_general(oh, a_ref[...], dn, preferred_element_type=jnp.float32)
    sc = lax.dot_general(oh, c_ref[...], dn, preferred_element_type=jnp.float32)
    ones = jnp.ones((bt.shape[0], 1), jnp.float32)
    ctc = lax.dot_general(oh, ones, dn, preferred_element_type=jnp.float32)
    ctr = jnp.sum(oh, axis=0, keepdims=True)

    rows = []
    for g in range(G):
        sel = jnp.where((bt == float(g)) & (incl > 0.0), 0.0, NEG)
        rows.append(jnp.max(x + sel, axis=0, keepdims=True))
    mx = jnp.concatenate(rows, axis=0)

    @pl.when(i == 0)
    def _():
        sx_ref[...] = sx
        sa_ref[...] = sa
        sc_ref[...] = sc
        ctc_ref[...] = ctc
        ctr_ref[...] = ctr
        mx_ref[...] = mx

    @pl.when(i > 0)
    def _():
        sx_ref[...] += sx
        sa_ref[...] += sa
        sc_ref[...] += sc
        ctc_ref[...] += ctc
        ctr_ref[...] += ctr
        mx_ref[...] = jnp.maximum(mx_ref[...], mx)


def _segreduce(x, a, c, bt, incl, block=1000):
    n, f = x.shape
    fa = a.shape[1]
    return pl.pallas_call(
        _seg_body,
        grid=(n // block,),
        in_specs=[pl.BlockSpec((block, f), lambda i: (i, 0)),
                  pl.BlockSpec((block, fa), lambda i: (i, 0)),
                  pl.BlockSpec((block, fa), lambda i: (i, 0)),
                  pl.BlockSpec((block, 1), lambda i: (i, 0)),
                  pl.BlockSpec((block, 1), lambda i: (i, 0))],
        out_specs=[pl.BlockSpec((G, f), lambda i: (0, 0)),
                   pl.BlockSpec((G, f), lambda i: (0, 0)),
                   pl.BlockSpec((G, 1), lambda i: (0, 0)),
                   pl.BlockSpec((1, G), lambda i: (0, 0)),
                   pl.BlockSpec((G, fa), lambda i: (0, 0)),
                   pl.BlockSpec((G, fa), lambda i: (0, 0))],
        out_shape=[jax.ShapeDtypeStruct((G, f), jnp.float32),
                   jax.ShapeDtypeStruct((G, f), jnp.float32),
                   jax.ShapeDtypeStruct((G, 1), jnp.float32),
                   jax.ShapeDtypeStruct((1, G), jnp.float32),
                   jax.ShapeDtypeStruct((G, fa), jnp.float32),
                   jax.ShapeDtypeStruct((G, fa), jnp.float32)],
    )(x, a, c, bt, incl)


def _topk_body(s_ref, bt_ref, act_ref, sT_ref, btT_ref, actT_ref, cnt_ref,
               mask_ref, ms_ref, *, block, cblock, npad):
    i = pl.program_id(0)
    s = s_ref[...]
    bt = bt_ref[...]
    act = act_ref[...]
    ridx = (i * block).astype(jnp.float32) + lax.broadcasted_iota(jnp.int32, (block, 1), 0).astype(jnp.float32)

    rank = jnp.zeros((block, 1), jnp.float32)
    for j in range(npad // cblock):
        cs = sT_ref[0:1, j * cblock:(j + 1) * cblock]
        cb = btT_ref[0:1, j * cblock:(j + 1) * cblock]
        ca = actT_ref[0:1, j * cblock:(j + 1) * cblock]
        cidx = float(j * cblock) + lax.broadcasted_iota(jnp.int32, (1, cblock), 1).astype(jnp.float32)
        beats = (cs > s) | ((cs == s) & (cidx < ridx))
        cmp = jnp.where((cb == bt) & (ca > 0.0) & beats, 1.0, 0.0)
        rank = rank + jnp.sum(cmp, axis=1, keepdims=True)

    gids = lax.broadcasted_iota(jnp.int32, (1, G), 1).astype(jnp.float32)
    k = jnp.ceil(RATIO * cnt_ref[...])
    krow = jnp.sum(jnp.where(bt == gids, k, 0.0), axis=1, keepdims=True)
    mask = jnp.where((act > 0.0) & (rank < krow), 1.0, 0.0)
    mask_ref[...] = mask
    ms_ref[...] = mask * s


def _topk(s, bt, act, sT, btT, actT, cnt, block=1000, cblock=2048):
    n = s.shape[0]
    npad = sT.shape[1]
    return pl.pallas_call(
        functools.partial(_topk_body, block=block, cblock=cblock, npad=npad),
        grid=(n // block,),
        in_specs=[pl.BlockSpec((block, 1), lambda i: (i, 0)),
                  pl.BlockSpec((block, 1), lambda i: (i, 0)),
                  pl.BlockSpec((block, 1), lambda i: (i, 0)),
                  pl.BlockSpec((1, npad), lambda i: (0, 0)),
                  pl.BlockSpec((1, npad), lambda i: (0, 0)),
                  pl.BlockSpec((1, npad), lambda i: (0, 0)),
                  pl.BlockSpec((1, G), lambda i: (0, 0))],
        out_specs=[pl.BlockSpec((block, 1), lambda i: (i, 0)),
                   pl.BlockSpec((block, 1), lambda i: (i, 0))],
        out_shape=[jax.ShapeDtypeStruct((n, 1), jnp.float32),
                   jax.ShapeDtypeStruct((n, 1), jnp.float32)],
    )(s, bt, act, sT, btT, actT, cnt)


def _fin_body(sx1, mx1, ct1, sx2, mx2, ct2, sx3, mx3, ct3, o_ref):
    f = sx1.shape[1]

    def gpool(sx_ref, mx_ref, ct_ref):
        cnt = ct_ref[...]
        mean = sx_ref[...] / jnp.maximum(cnt, 1.0)
        mx = jnp.where(cnt > 0.0, mx_ref[...], 0.0)
        return mx, mean
    a1, b1 = gpool(sx1, mx1, ct1)
    a2, b2 = gpool(sx2, mx2, ct2)
    a3, b3 = gpool(sx3, mx3, ct3)
    o_ref[:, :f] = a1 + a2 + a3
    o_ref[:, f:] = b1 + b2 + b3


def _finalize(parts):
    f = parts[0].shape[1]
    return pl.pallas_call(
        _fin_body,
        out_shape=jax.ShapeDtypeStruct((G, 2 * f), jnp.float32),
    )(*parts)


def _sc_mesh():
    return plsc.VectorSubcoreMesh(core_axis_name="c", subcore_axis_name="s",
                                  num_cores=NC, num_subcores=NS)


@functools.lru_cache(maxsize=None)
def _make_deg(n, e):
    ew = e // (NC * NS)

    def body(src_hbm, dst_hbm, m_hbm, d_out, m_v, d_v, s_v, t_v):
        wid = lax.axis_index("s") * NC + lax.axis_index("c")
        pltpu.sync_copy(m_hbm, m_v)
        pltpu.sync_copy(src_hbm.at[pl.ds(wid * ew, ew)], s_v)
        pltpu.sync_copy(dst_hbm.at[pl.ds(wid * ew, ew)], t_v)

        def zero(i, carry):
            d_v[pl.ds(i * LANES, LANES)] = jnp.zeros((LANES,), jnp.float32)
            return carry
        lax.fori_loop(0, n // LANES, zero, 0)

        def step(i, carry):
            sidx = s_v[pl.ds(i * LANES, LANES)]
            didx = t_v[pl.ds(i * LANES, LANES)]
            vals = plsc.load_gather(m_v, [sidx])
            plsc.addupdate_scatter(d_v, [didx], vals)
            return carry
        lax.fori_loop(0, ew // LANES, step, 0)
        pltpu.sync_copy(d_v, d_out.at[wid])

    return pl.kernel(
        body,
        out_type=jax.ShapeDtypeStruct((NC * NS, n), jnp.float32),
        mesh=_sc_mesh(),
        compiler_params=pltpu.CompilerParams(needs_layout_passes=False),
        scratch_types=[pltpu.VMEM((n,), jnp.float32),
                       pltpu.VMEM((n,), jnp.float32),
                       pltpu.VMEM((ew,), jnp.int32),
                       pltpu.VMEM((ew,), jnp.int32)],
    )


CHUNK = 125


@functools.lru_cache(maxsize=None)
def _make_spmm(n, e, h):
    chw = e // (NS * CHUNK)
    npad = _npad(n)
    stripe = npad // NS

    grp = 8

    def body(hs_hbm, src_hbm, dst_hbm, z_hbm, u_out, s_v, t_v,
             rows_a, rows_b, u_sh, sem):
        cid = lax.axis_index("c")
        sid = lax.axis_index("s")
        pltpu.sync_copy(z_hbm, u_sh.at[pl.ds(sid * stripe, stripe)])
        plsc.subcore_barrier()

        def group(j, carry):
            base = sid * chw + j * grp
            pltpu.sync_copy(src_hbm.at[cid].at[pl.ds(base, grp)], s_v)
            pltpu.sync_copy(dst_hbm.at[pl.ds(base, grp)], t_v)
            desc = pltpu.async_copy(hs_hbm.at[s_v.at[0]], rows_a, sem)
            for r in range(grp):
                buf = rows_a if r % 2 == 0 else rows_b
                nxt = rows_b if r % 2 == 0 else rows_a
                desc.wait()
                if r + 1 < grp:
                    desc = pltpu.async_copy(hs_hbm.at[s_v.at[r + 1]], nxt, sem)
                pltpu.sync_copy(buf, u_sh.at[t_v.at[r]], add=True)
            return carry
        lax.fori_loop(0, chw // grp, group, 0)
        plsc.subcore_barrier()
        pltpu.sync_copy(u_sh.at[pl.ds(sid * stripe, stripe)],
                        u_out.at[cid].at[pl.ds(sid * stripe, stripe)])

    return pl.kernel(
        body,
        out_type=jax.ShapeDtypeStruct((NC, npad, h), jnp.float32),
        mesh=_sc_mesh(),
        compiler_params=pltpu.CompilerParams(needs_layout_passes=False),
        scratch_types=[pltpu.VMEM((grp, CHUNK), jnp.int32),
                       pltpu.VMEM((grp, CHUNK), jnp.int32),
                       pltpu.VMEM((CHUNK, h), jnp.float32),
                       pltpu.VMEM((CHUNK, h), jnp.float32),
                       pltpu.VMEM_SHARED((npad, h), jnp.float32),
                       pltpu.SemaphoreType.DMA],
    )


def _npad(n):
    q = NS * 8
    return ((n + q - 1) // q) * q


def kernel(x, edge_index, batch, W1, b1, W2, b2, W3, b3, Wp1a, bp1a, Wp1b, bp1b,
           Wp2a, bp2a, Wp2b, bp2b, Wp3a, bp3a, Wp3b, bp3b, p1, p2):
    n = x.shape[0]
    e = edge_index.shape[1]
    hh = W1.shape[1]
    half = hh // 2

    src = edge_index[0]
    dst = edge_index[1]
    src2 = jnp.stack([src, src + n]).reshape(2, e // CHUNK, CHUNK)
    dst2 = dst.reshape(e // CHUNK, CHUNK)
    zrows = jnp.zeros((_npad(n) // NS, half), jnp.float32)
    ones_col = jnp.ones((n, 1), jnp.float32)
    btf = batch.astype(jnp.float32).reshape(n, 1)
    npad = ((n + 2047) // 2048) * 2048
    pad = npad - n

    def rowT(v, fill):
        return jnp.pad(v.reshape(1, n), ((0, 0), (0, pad)), constant_values=fill)
    btfT = rowT(btf, -1.0)
    onesT = rowT(ones_col, 0.0)

    deg_k = _make_deg(n, e)
    spmm_k = _make_spmm(n, e, half)

    def gcn_layer(feat, rowscale, m_col, W, b):
        m1d = m_col.reshape(n)
        dparts = deg_k(src, dst, m1d)
        hp = _mm(feat, W, jnp.zeros_like(b), rowscale=rowscale)
        hs2, dis = _scale(hp, dparts.T, m_col)
        u2 = spmm_k(hs2.reshape(2 * n, half), src2, dst2, zrows)
        return _assemble(u2, hp, dis, m_col, b)

    def mlp(v, wa, ba, wb, bb):
        return _mm(_mm(v, wa, ba, act=True), wb, bb)

    h1, x1 = gcn_layer(x, None, ones_col, W1, b1)
    seg1 = _segreduce(x1, mlp(h1, Wp1a, bp1a, Wp1b, bp1b),
                      mlp(x1, Wp1a, bp1a, Wp1b, bp1b), btf, ones_col)
    sx1, mxx1, ctc1, ctr1, g0_1, proj_1 = seg1
    s1 = _score(x1, p1)
    mask1, ms1 = _topk(s1, btf, ones_col, rowT(s1, 0.0), btfT, onesT, ctr1)

    h2, x2 = gcn_layer(x1, ms1, mask1, W2, b2)
    seg2 = _segreduce(x2, mlp(h2, Wp2a, bp2a, Wp2b, bp2b),
                      mlp(x2, Wp2a, bp2a, Wp2b, bp2b), btf, mask1)
    sx2, mxx2, ctc2, ctr2, g1_1, proj_2 = seg2
    s2 = _score(x2, p2)
    mask2, ms2 = _topk(s2, btf, mask1, rowT(s2, 0.0), btfT,
                       rowT(mask1, 0.0), ctr2)

    h3, x3 = gcn_layer(x2, ms2, mask2, W3, b3)
    seg3 = _segreduce(x3, mlp(h3, Wp3a, bp3a, Wp3b, bp3b),
                      mlp(x3, Wp3a, bp3a, Wp3b, bp3b), btf, mask2)
    sx3, mxx3, ctc3, ctr3, g2_1, proj_3 = seg3

    out = _finalize((sx1, mxx1, ctc1, sx2, mxx2, ctc2, sx3, mxx3, ctc3))
    return (out, proj_1, proj_2, proj_3, g0_1, g0_1, g1_1, g1_1, g2_1, g2_1)


# speedup vs baseline: 21.9856x; 1.3149x over previous
import functools
import jax
import jax.numpy as jnp
from jax import lax
from jax.experimental import pallas as pl
from jax.experimental.pallas import tpu as pltpu
from jax.experimental.pallas import tpu_sc as plsc

NC, NS, LANES = 2, 16, 16
G = 64
RATIO = 0.5
NEG = -3.0e38


def _bdot(a, w):
    return jnp.dot(a.astype(jnp.bfloat16), w.astype(jnp.bfloat16),
                   preferred_element_type=jnp.float32)


def _mm_body(a_ref, w_ref, b_ref, o_ref, *, act):
    h = _bdot(a_ref[...], w_ref[...]) + b_ref[...]
    if act:
        h = jnp.maximum(h, 0.0)
    o_ref[...] = h


def _mm_rs_body(a_ref, rs_ref, w_ref, b_ref, o_ref, *, act):
    a = a_ref[...] * rs_ref[...]
    h = _bdot(a, w_ref[...]) + b_ref[...]
    if act:
        h = jnp.maximum(h, 0.0)
    o_ref[...] = h


def _mm(a, w, b, act=False, rowscale=None, block=1000):
    n, k = a.shape
    m = w.shape[1]
    b2 = b.reshape(1, m)
    if rowscale is None:
        return pl.pallas_call(
            functools.partial(_mm_body, act=act),
            grid=(n // block,),
            in_specs=[pl.BlockSpec((block, k), lambda i: (i, 0)),
                      pl.BlockSpec((k, m), lambda i: (0, 0)),
                      pl.BlockSpec((1, m), lambda i: (0, 0))],
            out_specs=pl.BlockSpec((block, m), lambda i: (i, 0)),
            out_shape=jax.ShapeDtypeStruct((n, m), jnp.float32),
        )(a, w, b2)
    return pl.pallas_call(
        functools.partial(_mm_rs_body, act=act),
        grid=(n // block,),
        in_specs=[pl.BlockSpec((block, k), lambda i: (i, 0)),
                  pl.BlockSpec((block, 1), lambda i: (i, 0)),
                  pl.BlockSpec((k, m), lambda i: (0, 0)),
                  pl.BlockSpec((1, m), lambda i: (0, 0))],
        out_specs=pl.BlockSpec((block, m), lambda i: (i, 0)),
        out_shape=jax.ShapeDtypeStruct((n, m), jnp.float32),
    )(a, rowscale, w, b2)


def _scale_body(hp_ref, d_ref, m_ref, hs_ref, dis_ref):
    d = jnp.sum(d_ref[...], axis=1, keepdims=True)
    m = m_ref[...]
    dis = lax.rsqrt(1.0 + m * d)
    dis_ref[...] = dis
    hs = dis * hp_ref[...]
    h = hs_ref.shape[2]
    hs_ref[0, :, :] = hs[:, :h]
    hs_ref[1, :, :] = hs[:, h:]


def _scale(hp, dparts, m, block=1000):
    n, f = hp.shape
    h = f // 2
    return pl.pallas_call(
        _scale_body,
        grid=(n // block,),
        in_specs=[pl.BlockSpec((block, f), lambda i: (i, 0)),
                  pl.BlockSpec((block, NC * NS), lambda i: (i, 0)),
                  pl.BlockSpec((block, 1), lambda i: (i, 0))],
        out_specs=[pl.BlockSpec((2, block, h), lambda i: (0, i, 0)),
                   pl.BlockSpec((block, 1), lambda i: (i, 0))],
        out_shape=[jax.ShapeDtypeStruct((2, n, h), jnp.float32),
                   jax.ShapeDtypeStruct((n, 1), jnp.float32)],
    )(hp, dparts, m)


def _asm_body(u_ref, hp_ref, dis_ref, m_ref, b_ref, h_ref, x_ref):
    u = jnp.concatenate([u_ref[0], u_ref[1]], axis=1)
    dis = dis_ref[...]
    m = m_ref[...]
    h = (m * dis) * u + (dis * dis) * hp_ref[...] + b_ref[...]
    h_ref[...] = h
    x_ref[...] = jnp.maximum(h, 0.0)


def _assemble(u2, hp, dis, m, b, block=1000):
    n, f = hp.shape
    h = f // 2
    return pl.pallas_call(
        _asm_body,
        grid=(n // block,),
        in_specs=[pl.BlockSpec((2, block, h), lambda i: (0, i, 0)),
                  pl.BlockSpec((block, f), lambda i: (i, 0)),
                  pl.BlockSpec((block, 1), lambda i: (i, 0)),
                  pl.BlockSpec((block, 1), lambda i: (i, 0)),
                  pl.BlockSpec((1, f), lambda i: (0, 0))],
        out_specs=[pl.BlockSpec((block, f), lambda i: (i, 0)),
                   pl.BlockSpec((block, f), lambda i: (i, 0))],
        out_shape=[jax.ShapeDtypeStruct((n, f), jnp.float32),
                   jax.ShapeDtypeStruct((n, f), jnp.float32)],
    )(u2, hp, dis, m, b.reshape(1, f))


def _score_body(x_ref, p_ref, o_ref):
    p = p_ref[...]
    nrm = jnp.sqrt(jnp.sum(p * p))
    xb = x_ref[...].astype(jnp.bfloat16).astype(jnp.float32)
    pb = p.astype(jnp.bfloat16).astype(jnp.float32)
    s = jnp.sum(xb * pb, axis=1, keepdims=True) / nrm
    o_ref[...] = jnp.tanh(s)


def _score(x, p, block=1000):
    n, f = x.shape
    return pl.pallas_call(
        _score_body,
        grid=(n // block,),
        in_specs=[pl.BlockSpec((block, f), lambda i: (i, 0)),
                  pl.BlockSpec((1, f), lambda i: (0, 0))],
        out_specs=pl.BlockSpec((block, 1), lambda i: (i, 0)),
        out_shape=jax.ShapeDtypeStruct((n, 1), jnp.float32),
    )(x, p.reshape(1, f))


def _seg_body(x_ref, a_ref, c_ref, bt_ref, incl_ref,
              sx_ref, mx_ref, ctc_ref, ctr_ref, sa_ref, sc_ref):
    i = pl.program_id(0)
    bt = bt_ref[...]
    incl = incl_ref[...]
    gids = lax.broadcasted_iota(jnp.int32, (1, G), 1).astype(jnp.float32)
    oh = jnp.where(bt == gids, incl, 0.0)
    x = x_ref[...]
    dn = (((0,), (0,)), ((), ()))
    sx = lax.dot_general(oh, x, dn, preferred_element_type=jnp.float32)
    sa = lax.dot_general(oh, a_ref[...], dn, preferred_element_type=jnp.float32)
    sc = lax.dot_general(oh, c_ref[...], dn, preferred_element_type=jnp.float32)
    ones = jnp.ones((bt.shape[0], 1), jnp.float32)
    ctc = lax.dot_general(oh, ones, dn, preferred_element_type=jnp.float32)
    ctr = jnp.sum(oh, axis=0, keepdims=True)

    @pl.when(i == 0)
    def _():
        sx_ref[...] = jnp.zeros_like(sx_ref)
        sa_ref[...] = jnp.zeros_like(sa_ref)
        sc_ref[...] = jnp.zeros_like(sc_ref)
        ctc_ref[...] = jnp.zeros_like(ctc_ref)
        ctr_ref[...] = jnp.zeros_like(ctr_ref)
        mx_ref[...] = jnp.full_like(mx_ref, NEG)

    sx_ref[...] += sx
    sa_ref[...] += sa
    sc_ref[...] += sc
    ctc_ref[...] += ctc
    ctr_ref[...] += ctr

    lo = bt[0, 0]
    hi = bt[bt.shape[0] - 1, 0]
    for g in range(G):
        @pl.when((float(g) >= lo) & (float(g) <= hi))
        def _():
            sel = jnp.where((bt == float(g)) & (incl > 0.0), 0.0, NEG)
            mg = jnp.max(x + sel, axis=0, keepdims=True)
            mx_ref[g:g + 1, :] = jnp.maximum(mx_ref[g:g + 1, :], mg)


def _segreduce(x, a, c, bt, incl, block=1000):
    n, f = x.shape
    fa = a.shape[1]
    return pl.pallas_call(
        _seg_body,
        grid=(n // block,),
        in_specs=[pl.BlockSpec((block, f), lambda i: (i, 0)),
                  pl.BlockSpec((block, fa), lambda i: (i, 0)),
                  pl.BlockSpec((block, fa), lambda i: (i, 0)),
                  pl.BlockSpec((block, 1), lambda i: (i, 0)),
                  pl.BlockSpec((block, 1), lambda i: (i, 0))],
        out_specs=[pl.BlockSpec((G, f), lambda i: (0, 0)),
                   pl.BlockSpec((G, f), lambda i: (0, 0)),
                   pl.BlockSpec((G, 1), lambda i: (0, 0)),
                   pl.BlockSpec((1, G), lambda i: (0, 0)),
                   pl.BlockSpec((G, fa), lambda i: (0, 0)),
                   pl.BlockSpec((G, fa), lambda i: (0, 0))],
        out_shape=[jax.ShapeDtypeStruct((G, f), jnp.float32),
                   jax.ShapeDtypeStruct((G, f), jnp.float32),
                   jax.ShapeDtypeStruct((G, 1), jnp.float32),
                   jax.ShapeDtypeStruct((1, G), jnp.float32),
                   jax.ShapeDtypeStruct((G, fa), jnp.float32),
                   jax.ShapeDtypeStruct((G, fa), jnp.float32)],
    )(x, a, c, bt, incl)


def _topk_body(s_ref, bt_ref, act_ref, sT_ref, btT_ref, actT_ref, cnt_ref,
               mask_ref, ms_ref, rank_ref, *, block, cblock, npad):
    i = pl.program_id(0)
    s = s_ref[...]
    bt = bt_ref[...]
    act = act_ref[...]
    ridx = (i * block).astype(jnp.float32) + lax.broadcasted_iota(jnp.int32, (block, 1), 0).astype(jnp.float32)

    cb_all = btT_ref[...]
    lo = bt[0, 0]
    hi = bt[bt.shape[0] - 1, 0]
    lo_pos = jnp.sum(jnp.where(cb_all < lo, 1.0, 0.0))
    hi_pos = jnp.sum(jnp.where(cb_all <= hi, 1.0, 0.0))
    lo_c = jnp.floor(lo_pos / float(cblock))
    hi_c = jnp.floor((hi_pos - 1.0) / float(cblock))

    rank_ref[...] = jnp.zeros((block, 1), jnp.float32)
    for j in range(npad // cblock):
        @pl.when((float(j) >= lo_c) & (float(j) <= hi_c))
        def _():
            cs = sT_ref[0:1, j * cblock:(j + 1) * cblock]
            cb = btT_ref[0:1, j * cblock:(j + 1) * cblock]
            ca = actT_ref[0:1, j * cblock:(j + 1) * cblock]
            cidx = float(j * cblock) + lax.broadcasted_iota(jnp.int32, (1, cblock), 1).astype(jnp.float32)
            beats = (cs > s) | ((cs == s) & (cidx < ridx))
            cmp = jnp.where((cb == bt) & (ca > 0.0) & beats, 1.0, 0.0)
            rank_ref[...] += jnp.sum(cmp, axis=1, keepdims=True)
    rank = rank_ref[...]

    gids = lax.broadcasted_iota(jnp.int32, (1, G), 1).astype(jnp.float32)
    k = jnp.ceil(RATIO * cnt_ref[...])
    krow = jnp.sum(jnp.where(bt == gids, k, 0.0), axis=1, keepdims=True)
    mask = jnp.where((act > 0.0) & (rank < krow), 1.0, 0.0)
    mask_ref[...] = mask
    ms_ref[...] = mask * s


def _topk(s, bt, act, sT, btT, actT, cnt, block=1000, cblock=2048):
    n = s.shape[0]
    npad = sT.shape[1]
    return pl.pallas_call(
        functools.partial(_topk_body, block=block, cblock=cblock, npad=npad),
        grid=(n // block,),
        in_specs=[pl.BlockSpec((block, 1), lambda i: (i, 0)),
                  pl.BlockSpec((block, 1), lambda i: (i, 0)),
                  pl.BlockSpec((block, 1), lambda i: (i, 0)),
                  pl.BlockSpec((1, npad), lambda i: (0, 0)),
                  pl.BlockSpec((1, npad), lambda i: (0, 0)),
                  pl.BlockSpec((1, npad), lambda i: (0, 0)),
                  pl.BlockSpec((1, G), lambda i: (0, 0))],
        out_specs=[pl.BlockSpec((block, 1), lambda i: (i, 0)),
                   pl.BlockSpec((block, 1), lambda i: (i, 0))],
        out_shape=[jax.ShapeDtypeStruct((n, 1), jnp.float32),
                   jax.ShapeDtypeStruct((n, 1), jnp.float32)],
        scratch_shapes=[pltpu.VMEM((block, 1), jnp.float32)],
    )(s, bt, act, sT, btT, actT, cnt)


def _fin_body(sx1, mx1, ct1, sx2, mx2, ct2, sx3, mx3, ct3, o_ref):
    f = sx1.shape[1]

    def gpool(sx_ref, mx_ref, ct_ref):
        cnt = ct_ref[...]
        mean = sx_ref[...] / jnp.maximum(cnt, 1.0)
        mx = jnp.where(cnt > 0.0, mx_ref[...], 0.0)
        return mx, mean
    a1, b1 = gpool(sx1, mx1, ct1)
    a2, b2 = gpool(sx2, mx2, ct2)
    a3, b3 = gpool(sx3, mx3, ct3)
    o_ref[:, :f] = a1 + a2 + a3
    o_ref[:, f:] = b1 + b2 + b3


def _finalize(parts):
    f = parts[0].shape[1]
    return pl.pallas_call(
        _fin_body,
        out_shape=jax.ShapeDtypeStruct((G, 2 * f), jnp.float32),
    )(*parts)


def _sc_mesh():
    return plsc.VectorSubcoreMesh(core_axis_name="c", subcore_axis_name="s",
                                  num_cores=NC, num_subcores=NS)


@functools.lru_cache(maxsize=None)
def _make_deg(n, e):
    ew = e // (NC * NS)

    def body(src_hbm, dst_hbm, m_hbm, d_out, m_v, d_v, s_v, t_v):
        wid = lax.axis_index("s") * NC + lax.axis_index("c")
        pltpu.sync_copy(m_hbm, m_v)
        pltpu.sync_copy(src_hbm.at[pl.ds(wid * ew, ew)], s_v)
        pltpu.sync_copy(dst_hbm.at[pl.ds(wid * ew, ew)], t_v)

        def zero(i, carry):
            d_v[pl.ds(i * LANES, LANES)] = jnp.zeros((LANES,), jnp.float32)
            return carry
        lax.fori_loop(0, n // LANES, zero, 0)

        def step(i, carry):
            sidx = s_v[pl.ds(i * LANES, LANES)]
            didx = t_v[pl.ds(i * LANES, LANES)]
            vals = plsc.load_gather(m_v, [sidx])
            plsc.addupdate_scatter(d_v, [didx], vals)
            return carry
        lax.fori_loop(0, ew // LANES, step, 0)
        pltpu.sync_copy(d_v, d_out.at[wid])

    return pl.kernel(
        body,
        out_type=jax.ShapeDtypeStruct((NC * NS, n), jnp.float32),
        mesh=_sc_mesh(),
        compiler_params=pltpu.CompilerParams(needs_layout_passes=False),
        scratch_types=[pltpu.VMEM((n,), jnp.float32),
                       pltpu.VMEM((n,), jnp.float32),
                       pltpu.VMEM((ew,), jnp.int32),
                       pltpu.VMEM((ew,), jnp.int32)],
    )


CHUNK = 125


@functools.lru_cache(maxsize=None)
def _make_spmm(n, e, h):
    chw = e // (NS * CHUNK)
    npad = _npad(n)
    stripe = npad // NS

    grp = 8

    def body(hs_hbm, src_hbm, dst_hbm, z_hbm, u_out, s_v, t_v,
             rows_a, rows_b, u_sh, sem):
        cid = lax.axis_index("c")
        sid = lax.axis_index("s")
        pltpu.sync_copy(z_hbm, u_sh.at[pl.ds(sid * stripe, stripe)])
        plsc.subcore_barrier()

        def group(j, carry):
            base = sid * chw + j * grp
            pltpu.sync_copy(src_hbm.at[cid].at[pl.ds(base, grp)], s_v)
            pltpu.sync_copy(dst_hbm.at[pl.ds(base, grp)], t_v)
            desc = pltpu.async_copy(hs_hbm.at[s_v.at[0]], rows_a, sem)
            for r in range(grp):
                buf = rows_a if r % 2 == 0 else rows_b
                nxt = rows_b if r % 2 == 0 else rows_a
                desc.wait()
                if r + 1 < grp:
                    desc = pltpu.async_copy(hs_hbm.at[s_v.at[r + 1]], nxt, sem)
                pltpu.sync_copy(buf, u_sh.at[t_v.at[r]], add=True)
            return carry
        lax.fori_loop(0, chw // grp, group, 0)
        plsc.subcore_barrier()
        pltpu.sync_copy(u_sh.at[pl.ds(sid * stripe, stripe)],
                        u_out.at[cid].at[pl.ds(sid * stripe, stripe)])

    return pl.kernel(
        body,
        out_type=jax.ShapeDtypeStruct((NC, npad, h), jnp.float32),
        mesh=_sc_mesh(),
        compiler_params=pltpu.CompilerParams(needs_layout_passes=False),
        scratch_types=[pltpu.VMEM((grp, CHUNK), jnp.int32),
                       pltpu.VMEM((grp, CHUNK), jnp.int32),
                       pltpu.VMEM((CHUNK, h), jnp.float32),
                       pltpu.VMEM((CHUNK, h), jnp.float32),
                       pltpu.VMEM_SHARED((npad, h), jnp.float32),
                       pltpu.SemaphoreType.DMA],
    )


def _npad(n):
    q = NS * 8
    return ((n + q - 1) // q) * q


def kernel(x, edge_index, batch, W1, b1, W2, b2, W3, b3, Wp1a, bp1a, Wp1b, bp1b,
           Wp2a, bp2a, Wp2b, bp2b, Wp3a, bp3a, Wp3b, bp3b, p1, p2):
    n = x.shape[0]
    e = edge_index.shape[1]
    hh = W1.shape[1]
    half = hh // 2

    src = edge_index[0]
    dst = edge_index[1]
    src2 = jnp.stack([src, src + n]).reshape(2, e // CHUNK, CHUNK)
    dst2 = dst.reshape(e // CHUNK, CHUNK)
    zrows = jnp.zeros((_npad(n) // NS, half), jnp.float32)
    ones_col = jnp.ones((n, 1), jnp.float32)
    btf = batch.astype(jnp.float32).reshape(n, 1)
    npad = ((n + 2047) // 2048) * 2048
    pad = npad - n

    def rowT(v, fill):
        return jnp.pad(v.reshape(1, n), ((0, 0), (0, pad)), constant_values=fill)
    btfT = rowT(btf, 1.0e9)
    onesT = rowT(ones_col, 0.0)

    deg_k = _make_deg(n, e)
    spmm_k = _make_spmm(n, e, half)

    def gcn_layer(feat, rowscale, m_col, W, b):
        m1d = m_col.reshape(n)
        dparts = deg_k(src, dst, m1d)
        hp = _mm(feat, W, jnp.zeros_like(b), rowscale=rowscale)
        hs2, dis = _scale(hp, dparts.T, m_col)
        u2 = spmm_k(hs2.reshape(2 * n, half), src2, dst2, zrows)
        return _assemble(u2, hp, dis, m_col, b)

    def mlp(v, wa, ba, wb, bb):
        return _mm(_mm(v, wa, ba, act=True), wb, bb)

    h1, x1 = gcn_layer(x, None, ones_col, W1, b1)
    seg1 = _segreduce(x1, mlp(h1, Wp1a, bp1a, Wp1b, bp1b),
                      mlp(x1, Wp1a, bp1a, Wp1b, bp1b), btf, ones_col)
    sx1, mxx1, ctc1, ctr1, g0_1, proj_1 = seg1
    s1 = _score(x1, p1)
    mask1, ms1 = _topk(s1, btf, ones_col, rowT(s1, 0.0), btfT, onesT, ctr1)

    h2, x2 = gcn_layer(x1, ms1, mask1, W2, b2)
    seg2 = _segreduce(x2, mlp(h2, Wp2a, bp2a, Wp2b, bp2b),
                      mlp(x2, Wp2a, bp2a, Wp2b, bp2b), btf, mask1)
    sx2, mxx2, ctc2, ctr2, g1_1, proj_2 = seg2
    s2 = _score(x2, p2)
    mask2, ms2 = _topk(s2, btf, mask1, rowT(s2, 0.0), btfT,
                       rowT(mask1, 0.0), ctr2)

    h3, x3 = gcn_layer(x2, ms2, mask2, W3, b3)
    seg3 = _segreduce(x3, mlp(h3, Wp3a, bp3a, Wp3b, bp3b),
                      mlp(x3, Wp3a, bp3a, Wp3b, bp3b), btf, mask2)
    sx3, mxx3, ctc3, ctr3, g2_1, proj_3 = seg3

    out = _finalize((sx1, mxx1, ctc1, sx2, mxx2, ctc2, sx3, mxx3, ctc3))
    return (out, proj_1, proj_2, proj_3, g0_1, g0_1, g1_1, g1_1, g2_1, g2_1)
